# Initial kernel scaffold; baseline (speedup 1.0000x reference)
import jax
import jax.numpy as jnp
from jax.experimental import pallas as pl


def kernel(x, gauges, edge_index, kernels, W1, b1, W2, b2, We1, be1, We2, be2):
    raise NotImplementedError("write your pallas kernel here")


# speedup vs baseline: 1.8778x; 1.8778x over previous
import functools

import jax
import jax.numpy as jnp
from jax import lax
from jax.experimental import pallas as pl
from jax.experimental.pallas import tpu as pltpu
from jax.experimental.pallas import tpu_sc as plsc

N = 50000
E = 800000
E_PAD = 819200
CH = 128
KB = 5
NW = 32
EPW = E_PAD // NW
EPT = E_PAD // 16
HALF = N // 2
SROW = 1568
TROWS = 16 * SROW
NDUMP = 64
NB = 1000
EB = 8192
F32 = jnp.float32


def _mesh():
    return plsc.VectorSubcoreMesh(core_axis_name="c", subcore_axis_name="s")


@functools.lru_cache(maxsize=None)
def _make_gather(width):
    nit = EPW // (KB * CH)

    @functools.partial(
        pl.kernel,
        out_type=jax.ShapeDtypeStruct((E_PAD, width), F32),
        mesh=_mesh(),
        compiler_params=pltpu.CompilerParams(use_tc_tiling_on_sc=False),
        scratch_types=[
            pltpu.VMEM((KB, CH), jnp.int32),
            pltpu.VMEM((KB * CH, width), F32),
            pltpu.SemaphoreType.DMA,
        ],
    )
    def gather_k(z_hbm, src_hbm, g_hbm, idx_v, rows_v, sem):
        c = lax.axis_index("c")
        s = lax.axis_index("s")
        w = s * 2 + c

        def body(i, carry):
            it = w * nit + i
            blk = it * KB
            pltpu.sync_copy(src_hbm.at[it], idx_v)
            cps = [
                pltpu.async_copy(
                    z_hbm.at[idx_v.at[j]],
                    rows_v.at[pl.ds(j * CH, CH), :],
                    sem,
                )
                for j in range(KB)
            ]
            for cp in cps:
                cp.wait()
            pltpu.sync_copy(rows_v, g_hbm.at[pl.ds(blk * CH, KB * CH), :])
            return carry

        lax.fori_loop(0, nit, body, 0)

    return gather_k


@functools.lru_cache(maxsize=None)
def _make_scatter():
    nit = EPT // (KB * CH)

    @functools.partial(
        pl.kernel,
        out_type=jax.ShapeDtypeStruct((N, 48), F32),
        mesh=_mesh(),
        compiler_params=pltpu.CompilerParams(use_tc_tiling_on_sc=False),
        scratch_types=[
            pltpu.VMEM((KB, CH), jnp.int32),
            pltpu.VMEM((KB * CH, 48), F32),
            pltpu.VMEM_SHARED((TROWS, 48), F32),
            pltpu.SemaphoreType.DMA,
        ],
    )
    def scatter_k(u_hbm, loc_hbm, zeros_hbm, out_hbm, idx_v, rows_v, table_sh, sem):
        c = lax.axis_index("c")
        s = lax.axis_index("s")
        pltpu.sync_copy(zeros_hbm, table_sh.at[pl.ds(s * SROW, SROW), :])
        plsc.subcore_barrier()

        def body(i, carry):
            it = s * nit + i
            blk = it * KB
            pltpu.sync_copy(loc_hbm.at[c, it], idx_v)
            pltpu.sync_copy(u_hbm.at[pl.ds(blk * CH, KB * CH), :], rows_v)
            cps = [
                pltpu.async_copy(
                    rows_v.at[pl.ds(j * CH, CH), :],
                    table_sh.at[idx_v.at[j]],
                    sem,
                    add=True,
                )
                for j in range(KB)
            ]
            for cp in cps:
                cp.wait()
            return carry

        lax.fori_loop(0, nit, body, 0)
        plsc.subcore_barrier()

        @pl.when(s < 15)
        def _copy_full():
            pltpu.sync_copy(
                table_sh.at[pl.ds(s * SROW, SROW), :],
                out_hbm.at[pl.ds(c * HALF + s * SROW, SROW), :],
            )

        @pl.when(s == 15)
        def _copy_tail():
            pltpu.sync_copy(
                table_sh.at[pl.ds(15 * SROW, HALF - 15 * SROW), :],
                out_hbm.at[pl.ds(c * HALF + 15 * SROW, HALF - 15 * SROW), :],
            )

    return scatter_k


def _full(shape):
    return pl.BlockSpec(shape, lambda i: (0,) * len(shape))


def _rows(shape):
    return pl.BlockSpec(shape, lambda i: (i,) + (0,) * (len(shape) - 1))


def _prep_body(x_ref, g9_ref, w_ref, hpad_ref, self1_ref):
    xc = [x_ref[:, d:d + 1] for d in range(3)]
    h = [
        g9_ref[:, 3 * i:3 * i + 1] * xc[0]
        + g9_ref[:, 3 * i + 1:3 * i + 2] * xc[1]
        + g9_ref[:, 3 * i + 2:3 * i + 3] * xc[2]
        for i in range(3)
    ]
    nrm = jnp.sqrt(h[0] * h[0] + h[1] * h[1] + h[2] * h[2])
    inv = 1.0 / (nrm + 1e-8)
    hn = [hi * inv for hi in h]
    hpad_ref[...] = jnp.concatenate(
        hn + [jnp.zeros((hpad_ref.shape[0], 13), F32)], axis=1)
    self1_ref[...] = (hn[0] * w_ref[0:1, :] + hn[1] * w_ref[1:2, :]
                      + hn[2] * w_ref[2:3, :])


def _combine1_body(hg_ref, k_ref, u_ref):
    u_ref[...] = jnp.concatenate(
        [k_ref[:, j:j + 1] * hg_ref[...] for j in range(3)], axis=1)


def _combine2_body(zg_ref, k_ref, u_ref):
    acc = k_ref[:, 0:1] * zg_ref[:, 0:48]
    acc += k_ref[:, 1:2] * zg_ref[:, 48:96]
    acc += k_ref[:, 2:3] * zg_ref[:, 96:144]
    u_ref[...] = acc


def _remap_body(dst_ref, loc0_ref, loc1_ref):
    d = dst_ref[...]
    lane = lax.broadcasted_iota(jnp.int32, d.shape, 1)
    dump = HALF + (lane & (NDUMP - 1))
    loc0_ref[...] = jnp.where(d < HALF, d, dump)
    loc1_ref[...] = jnp.where((d >= HALF) & (d < N), d - HALF, dump)


def _layer1_body(m1_ref, self1_ref, w1_ref, b1_ref, wcat2_ref, w2s_ref,
                 h1_ref, z2_ref, self2_ref):
    pre = self1_ref[...] + b1_ref[...]
    for j in range(3):
        for dd in range(3):
            pre += (m1_ref[:, 16 * j + dd:16 * j + dd + 1]
                    * w1_ref[3 + 3 * j + dd:4 + 3 * j + dd, :])
    h1 = jnp.tanh(pre)
    h1_ref[...] = h1
    z2_ref[...] = jnp.dot(h1, wcat2_ref[...], preferred_element_type=F32)
    self2_ref[...] = jnp.dot(h1, w2s_ref[...], preferred_element_type=F32)


def _final_body(hpad_ref, h1_ref, self2_ref, m2_ref, b2_ref,
                we1h_ref, we1a_ref, we1b_ref, be1_ref, we2_ref, be2_ref,
                emb_ref):
    h2 = jnp.tanh(self2_ref[...] + m2_ref[...] + b2_ref[...])
    h1 = h1_ref[...]
    t = be1_ref[...] + jnp.dot(h1, we1a_ref[...], preferred_element_type=F32)
    t += jnp.dot(h2, we1b_ref[...], preferred_element_type=F32)
    for dd in range(3):
        t += hpad_ref[:, dd:dd + 1] * we1h_ref[dd:dd + 1, :]
    a = jnp.maximum(t, 0.0)
    emb_ref[...] = jnp.dot(a, we2_ref[...], preferred_element_type=F32) + be2_ref[...]


def kernel(x, gauges, edge_index, kernels, W1, b1, W2, b2, We1, be1, We2, be2):
    src = edge_index[0]
    dst = edge_index[1]
    g9 = gauges.reshape(N, 9)
    ktr = jnp.pad(kernels, ((0, 0), (0, E_PAD - E))).T
    src2 = jnp.pad(src, (0, E_PAD - E)).reshape(E_PAD // (KB * CH), KB, CH)
    dst2 = jnp.pad(dst, (0, E_PAD - E), constant_values=1 << 29)
    dst2 = dst2.reshape(400, 2048)
    zeros_tab = jnp.zeros((SROW, 48), F32)
    Wcat2 = jnp.concatenate([W2[48:96], W2[96:144], W2[144:192]], axis=1)
    b1r = b1.reshape(1, 48)
    b2r = b2.reshape(1, 48)
    be1r = be1.reshape(1, 128)
    be2r = be2.reshape(1, 64)
    grid_n = N // NB
    grid_e = E_PAD // EB

    hpad, self1 = pl.pallas_call(
        _prep_body,
        grid=(grid_n,),
        in_specs=[_rows((NB, 3)), _rows((NB, 9)), _full((3, 48))],
        out_specs=[_rows((NB, 16)), _rows((NB, 48))],
        out_shape=[jax.ShapeDtypeStruct((N, 16), F32),
                   jax.ShapeDtypeStruct((N, 48), F32)],
    )(x, g9, W1[0:3])

    loc0, loc1 = pl.pallas_call(
        _remap_body,
        grid=(50,),
        in_specs=[_rows((8, 2048))],
        out_specs=[_rows((8, 2048)), _rows((8, 2048))],
        out_shape=[jax.ShapeDtypeStruct((400, 2048), jnp.int32),
                   jax.ShapeDtypeStruct((400, 2048), jnp.int32)],
    )(dst2)
    loc = jnp.stack([loc0.reshape(E_PAD // (KB * CH), KB, CH),
                     loc1.reshape(E_PAD // (KB * CH), KB, CH)])

    hg = _make_gather(16)(hpad, src2)

    u1 = pl.pallas_call(
        _combine1_body,
        grid=(grid_e,),
        in_specs=[_rows((EB, 16)), _rows((EB, 3))],
        out_specs=_rows((EB, 48)),
        out_shape=jax.ShapeDtypeStruct((E_PAD, 48), F32),
    )(hg, ktr)

    m1 = _make_scatter()(u1, loc, zeros_tab)

    h1, z2, self2 = pl.pallas_call(
        _layer1_body,
        grid=(grid_n,),
        in_specs=[_rows((NB, 48)), _rows((NB, 48)), _full((12, 48)),
                  _full((1, 48)), _full((48, 144)), _full((48, 48))],
        out_specs=[_rows((NB, 48)), _rows((NB, 144)), _rows((NB, 48))],
        out_shape=[jax.ShapeDtypeStruct((N, 48), F32),
                   jax.ShapeDtypeStruct((N, 144), F32),
                   jax.ShapeDtypeStruct((N, 48), F32)],
    )(m1, self1, W1, b1r, Wcat2, W2[0:48])

    zg = _make_gather(144)(z2, src2)

    u2 = pl.pallas_call(
        _combine2_body,
        grid=(grid_e,),
        in_specs=[_rows((EB, 144)), _rows((EB, 3))],
        out_specs=_rows((EB, 48)),
        out_shape=jax.ShapeDtypeStruct((E_PAD, 48), F32),
    )(zg, ktr)

    m2 = _make_scatter()(u2, loc, zeros_tab)

    emb = pl.pallas_call(
        _final_body,
        grid=(grid_n,),
        in_specs=[_rows((NB, 16)), _rows((NB, 48)), _rows((NB, 48)),
                  _rows((NB, 48)), _full((1, 48)),
                  _full((3, 128)), _full((48, 128)), _full((48, 128)),
                  _full((1, 128)), _full((128, 64)), _full((1, 64))],
        out_specs=_rows((NB, 64)),
        out_shape=jax.ShapeDtypeStruct((N, 64), F32),
    )(hpad, h1, self2, m2, b2r,
      We1[0:3], We1[3:51], We1[51:99], be1r, We2, be2r)

    return emb


# speedup vs baseline: 6.0295x; 3.2110x over previous
import functools

import jax
import jax.numpy as jnp
from jax import lax
from jax.experimental import pallas as pl
from jax.experimental.pallas import tpu as pltpu
from jax.experimental.pallas import tpu_sc as plsc

N = 50000
E = 800000
E_PAD = 819200
CH = 128
GRP = 20
TR = 50048
ZR = TR // 16
NDUMP = 32
NB = 1000
F32 = jnp.float32


def _mesh():
    return plsc.VectorSubcoreMesh(core_axis_name="c", subcore_axis_name="s")


@functools.lru_cache(maxsize=None)
def _make_fused(zwidth, l1):
    ept = (E_PAD // 32) if l1 else (E_PAD // 16)
    nitf = ept // CH
    ngrp = nitf // GRP
    pairs = GRP // 2

    @functools.partial(
        pl.kernel,
        out_type=jax.ShapeDtypeStruct((2, TR, 24), F32),
        mesh=_mesh(),
        compiler_params=pltpu.CompilerParams(use_tc_tiling_on_sc=False),
        scratch_types=[
            pltpu.VMEM((GRP, CH), jnp.int32),
            pltpu.VMEM((GRP, CH), jnp.int32),
            pltpu.VMEM((3, GRP * CH + 16), F32),
            pltpu.VMEM((2, CH, zwidth), F32),
            pltpu.VMEM((2, CH, 24), F32),
            pltpu.VMEM_SHARED((TR, 24), F32),
            pltpu.SemaphoreType.DMA,
            pltpu.SemaphoreType.DMA,
            pltpu.SemaphoreType.DMA,
        ],
    )
    def fused_k(ztab, src_hbm, dst_hbm, kern_hbm, zeros_hbm, m_hbm,
                isrc_v, idst_v, kv_v, rows_v, u_v, table_sh,
                semg, sems0, sems1):
        c = lax.axis_index("c")
        s = lax.axis_index("s")
        w = s * 2 + c

        pltpu.sync_copy(zeros_hbm, table_sh.at[pl.ds(s * ZR, ZR), :])
        plsc.subcore_barrier()

        def g_descs(r, buf):
            return [pltpu.make_async_copy(
                ztab.at[isrc_v.at[r]], rows_v.at[buf], semg)]

        def s_descs(r, buf, sem):
            return [pltpu.make_async_copy(
                u_v.at[buf], table_sh.at[idst_v.at[r]], sem)]

        def combine(r, buf):
            base = r * CH

            def edge_body(e, carry):
                k1 = kv_v[0, pl.ds(base + e, 16)][0]
                k2 = kv_v[1, pl.ds(base + e, 16)][0]
                k3 = kv_v[2, pl.ds(base + e, 16)][0]
                if l1:
                    lane = lax.broadcasted_iota(jnp.int32, (16,), 0)
                    lo = lane < 8
                    r16 = rows_v[buf, e, pl.ds(0, 16)]
                    u_v[buf, e, pl.ds(0, 16)] = jnp.where(lo, k1, k2) * r16
                    u_v[buf, e, pl.ds(8, 16)] = (
                        jnp.where(lo, k2, k3)
                        * rows_v[buf, e, pl.ds(0, 16)])
                else:
                    acc_a = k1 * rows_v[buf, e, pl.ds(0, 16)]
                    acc_a += k2 * rows_v[buf, e, pl.ds(32, 16)]
                    acc_a += k3 * rows_v[buf, e, pl.ds(64, 16)]
                    u_v[buf, e, pl.ds(0, 16)] = acc_a
                    acc_b = k1 * rows_v[buf, e, pl.ds(8, 16)]
                    acc_b += k2 * rows_v[buf, e, pl.ds(40, 16)]
                    acc_b += k3 * rows_v[buf, e, pl.ds(72, 16)]
                    u_v[buf, e, pl.ds(8, 16)] = acc_b
                return carry

            lax.fori_loop(0, CH, edge_body, 0)

        def group_body(g, carry):
            if l1:
                blk = w * nitf + g * GRP
                off = w * ept + g * (GRP * CH)
                pltpu.sync_copy(src_hbm.at[0, pl.ds(blk, GRP), :], isrc_v)
            else:
                blk = s * nitf + g * GRP
                off = s * ept + g * (GRP * CH)
                pltpu.sync_copy(src_hbm.at[c, pl.ds(blk, GRP), :], isrc_v)
            pltpu.sync_copy(dst_hbm.at[pl.ds(blk, GRP), :], idst_v)
            for j in range(3):
                pltpu.sync_copy(kern_hbm.at[j, pl.ds(off, GRP * CH)],
                                kv_v.at[j, pl.ds(0, GRP * CH)])
            for d in g_descs(0, 0):
                d.start()

            def pair_body(p, carry2):
                a = 2 * p
                b = 2 * p + 1
                for d in g_descs(a, 0):
                    d.wait()

                @pl.when(p > 0)
                def _():
                    for d in s_descs(a - 2, 0, sems0):
                        d.wait()

                for d in g_descs(b, 1):
                    d.start()
                combine(a, 0)
                for d in s_descs(a, 0, sems0):
                    d.start(add=True)
                for d in g_descs(b, 1):
                    d.wait()

                @pl.when(p > 0)
                def _():
                    for d in s_descs(b - 2, 1, sems1):
                        d.wait()

                @pl.when(p < pairs - 1)
                def _():
                    for d in g_descs(b + 1, 0):
                        d.start()

                combine(b, 1)
                for d in s_descs(b, 1, sems1):
                    d.start(add=True)
                return carry2

            lax.fori_loop(0, pairs, pair_body, 0)
            for d in s_descs(GRP - 2, 0, sems0):
                d.wait()
            for d in s_descs(GRP - 1, 1, sems1):
                d.wait()
            return carry

        lax.fori_loop(0, ngrp, group_body, 0)
        plsc.subcore_barrier()
        pltpu.sync_copy(table_sh.at[pl.ds(s * ZR, ZR), :],
                        m_hbm.at[c, pl.ds(s * ZR, ZR), :])

    return fused_k


def _full(shape):
    return pl.BlockSpec(shape, lambda i: (0,) * len(shape))


def _rows(shape):
    return pl.BlockSpec(shape, lambda i: (i,) + (0,) * (len(shape) - 1))


def _rows1(shape):
    return pl.BlockSpec(shape, lambda i: (0, i) + (0,) * (len(shape) - 2))


def _prep_body(x_ref, g9_ref, w_ref, htab_ref, self1_ref):
    xc = [x_ref[:, d:d + 1] for d in range(3)]
    h = [
        g9_ref[:, 3 * i:3 * i + 1] * xc[0]
        + g9_ref[:, 3 * i + 1:3 * i + 2] * xc[1]
        + g9_ref[:, 3 * i + 2:3 * i + 3] * xc[2]
        for i in range(3)
    ]
    nrm = jnp.sqrt(h[0] * h[0] + h[1] * h[1] + h[2] * h[2])
    inv = 1.0 / (nrm + 1e-8)
    hn = [hi * inv for hi in h]
    z5 = jnp.zeros((x_ref.shape[0], 5), F32)
    htab_ref[...] = jnp.concatenate(hn + [z5] + hn + [z5], axis=1)
    self1_ref[...] = (hn[0] * w_ref[0:1, :] + hn[1] * w_ref[1:2, :]
                      + hn[2] * w_ref[2:3, :])


def _layer1_body(m0_ref, m1_ref, self1_ref, w1_ref, b1_ref, wz_ref, w2s_ref,
                 h1_ref, ztab_ref, self2_ref):
    pre = self1_ref[...] + b1_ref[...]
    m = m0_ref[...] + m1_ref[...]
    for j in range(3):
        for d in range(3):
            pre += (m[:, 8 * j + d:8 * j + d + 1]
                    * w1_ref[3 + 3 * j + d:4 + 3 * j + d, :])
    h1 = jnp.tanh(pre)
    h1_ref[...] = h1
    z = jnp.dot(h1, wz_ref[...], preferred_element_type=F32)
    ztab_ref[0] = z[:, 0:96]
    ztab_ref[1] = z[:, 96:192]
    self2_ref[...] = jnp.dot(h1, w2s_ref[...], preferred_element_type=F32)


def _t0_body(htab_ref, we1h_ref, be1_ref, t0_ref):
    t = be1_ref[...]
    for d in range(3):
        t = t + htab_ref[:, d:d + 1] * we1h_ref[d:d + 1, :]
    t0_ref[...] = t


def _final_body(h1_ref, self2_ref, ma_ref, mb_ref, b2_ref,
                we1a_ref, we1b_ref, t0_ref, we2_ref, be2_ref, emb_ref):
    msum = jnp.concatenate([ma_ref[...], mb_ref[...]], axis=1)
    h2 = jnp.tanh(self2_ref[...] + msum + b2_ref[...])
    h1 = h1_ref[...]
    t = t0_ref[...] + jnp.dot(h1, we1a_ref[...], preferred_element_type=F32)
    t += jnp.dot(h2, we1b_ref[...], preferred_element_type=F32)
    a = jnp.maximum(t, 0.0)
    emb_ref[...] = (jnp.dot(a, we2_ref[...], preferred_element_type=F32)
                    + be2_ref[...])


def kernel(x, gauges, edge_index, kernels, W1, b1, W2, b2, We1, be1, We2, be2):
    src = edge_index[0]
    dst = edge_index[1]
    g9 = gauges.reshape(N, 9)
    srcp = jnp.pad(src, (0, E_PAD - E))
    src2 = jnp.stack([srcp, srcp + N]).reshape(2, E_PAD // CH, CH)
    pad_dump = N + (jnp.arange(E_PAD - E, dtype=jnp.int32) % NDUMP)
    dstp = jnp.concatenate([dst, pad_dump]).reshape(E_PAD // CH, CH)
    kpad = jnp.pad(kernels, ((0, 0), (0, E_PAD - E)))
    zeros_tab = jnp.zeros((ZR, 24), F32)
    wz_cols = []
    for h in range(2):
        for j in range(3):
            blk = W2[48 * (j + 1):48 * (j + 2), 24 * h:24 * h + 24]
            wz_cols.append(blk)
            wz_cols.append(jnp.zeros((48, 8), F32))
    Wz = jnp.concatenate(wz_cols, axis=1)
    b1r = b1.reshape(1, 48)
    b2r = b2.reshape(1, 48)
    grid_n = N // NB

    htab, self1 = pl.pallas_call(
        _prep_body,
        grid=(grid_n,),
        in_specs=[_rows((NB, 3)), _rows((NB, 9)), _full((3, 48))],
        out_specs=[_rows((NB, 16)), _rows((NB, 48))],
        out_shape=[jax.ShapeDtypeStruct((N, 16), F32),
                   jax.ShapeDtypeStruct((N, 48), F32)],
    )(x, g9, W1[0:3])

    m1 = _make_fused(16, True)(htab, src2, dstp, kpad, zeros_tab)

    h1, ztab, self2 = pl.pallas_call(
        _layer1_body,
        grid=(grid_n,),
        in_specs=[_rows((NB, 24)), _rows((NB, 24)), _rows((NB, 48)),
                  _full((12, 48)), _full((1, 48)), _full((48, 192)),
                  _full((48, 48))],
        out_specs=[_rows((NB, 48)), _rows1((2, NB, 96)), _rows((NB, 48))],
        out_shape=[jax.ShapeDtypeStruct((N, 48), F32),
                   jax.ShapeDtypeStruct((2, N, 96), F32),
                   jax.ShapeDtypeStruct((N, 48), F32)],
    )(m1[0, :N], m1[1, :N], self1, W1, b1r, Wz, W2[0:48])

    m2 = _make_fused(96, False)(ztab.reshape(2 * N, 96), src2, dstp, kpad,
                                zeros_tab)

    t0 = pl.pallas_call(
        _t0_body,
        grid=(grid_n,),
        in_specs=[_rows((NB, 16)), _full((3, 128)), _full((1, 128))],
        out_specs=_rows((NB, 128)),
        out_shape=jax.ShapeDtypeStruct((N, 128), F32),
    )(htab, We1[0:3], be1.reshape(1, 128))

    emb = pl.pallas_call(
        _final_body,
        grid=(grid_n,),
        in_specs=[_rows((NB, 48)), _rows((NB, 48)), _rows((NB, 24)),
                  _rows((NB, 24)), _full((1, 48)),
                  _full((48, 128)), _full((48, 128)), _rows((NB, 128)),
                  _full((128, 64)), _full((1, 64))],
        out_specs=_rows((NB, 64)),
        out_shape=jax.ShapeDtypeStruct((N, 64), F32),
    )(h1, self2, m2[0, :N], m2[1, :N], b2r,
      We1[3:51], We1[51:99], t0, We2, be2.reshape(1, 64))

    return emb


# speedup vs baseline: 7.1499x; 1.1858x over previous
import functools

import jax
import jax.numpy as jnp
from jax import lax
from jax.experimental import pallas as pl
from jax.experimental.pallas import tpu as pltpu
from jax.experimental.pallas import tpu_sc as plsc

N = 50000
E = 800000
E_PAD = 819200
CH = 128
GRP = 10
TR = 50048
ZR = TR // 16
NDUMP = 32
NB = 5000
F32 = jnp.float32


def _mesh():
    return plsc.VectorSubcoreMesh(core_axis_name="c", subcore_axis_name="s")


@functools.lru_cache(maxsize=None)
def _make_fused(zwidth, l1):
    ept = (E_PAD // 32) if l1 else (E_PAD // 16)
    nitf = ept // CH
    ngrp = nitf // GRP
    nbuf = 3

    @functools.partial(
        pl.kernel,
        out_type=jax.ShapeDtypeStruct((2, TR, 24), F32),
        mesh=_mesh(),
        compiler_params=pltpu.CompilerParams(use_tc_tiling_on_sc=False),
        scratch_types=[
            pltpu.VMEM((GRP, CH), jnp.int32),
            pltpu.VMEM((GRP, CH), jnp.int32),
            pltpu.VMEM((3, GRP * CH + 16), F32),
            pltpu.VMEM((nbuf, CH, zwidth), F32),
            pltpu.VMEM((nbuf, CH, 24), F32),
            pltpu.VMEM_SHARED((TR, 24), F32),
            [pltpu.SemaphoreType.DMA] * nbuf,
            [pltpu.SemaphoreType.DMA] * nbuf,
        ],
    )
    def fused_k(ztab, src_hbm, dst_hbm, kern_hbm, zeros_hbm, m_hbm,
                isrc_v, idst_v, kv_v, rows_v, u_v, table_sh,
                semg, sems):
        c = lax.axis_index("c")
        s = lax.axis_index("s")
        w = s * 2 + c

        pltpu.sync_copy(zeros_hbm, table_sh.at[pl.ds(s * ZR, ZR), :])
        plsc.subcore_barrier()

        def g_desc(r):
            b = r % nbuf
            return pltpu.make_async_copy(
                ztab.at[isrc_v.at[r]], rows_v.at[b], semg[b])

        def s_desc(r):
            b = r % nbuf
            return pltpu.make_async_copy(
                u_v.at[b], table_sh.at[idst_v.at[r]], sems[b])

        def combine(r):
            buf = r % nbuf
            base = r * CH

            def edge_body(t, carry):
                e0 = t * 4
                for q in range(4):
                    e = e0 + q
                    k1 = kv_v[0, pl.ds(base + e, 16)][0]
                    k2 = kv_v[1, pl.ds(base + e, 16)][0]
                    k3 = kv_v[2, pl.ds(base + e, 16)][0]
                    if l1:
                        lane = lax.broadcasted_iota(jnp.int32, (16,), 0)
                        lo = lane < 8
                        r16 = rows_v[buf, e, pl.ds(0, 16)]
                        u_v[buf, e, pl.ds(0, 16)] = (
                            jnp.where(lo, k1, k2) * r16)
                        u_v[buf, e, pl.ds(8, 16)] = (
                            jnp.where(lo, k2, k3) * r16)
                    else:
                        acc_a = k1 * rows_v[buf, e, pl.ds(0, 16)]
                        acc_a += k2 * rows_v[buf, e, pl.ds(32, 16)]
                        acc_a += k3 * rows_v[buf, e, pl.ds(64, 16)]
                        u_v[buf, e, pl.ds(0, 16)] = acc_a
                        acc_b = k1 * rows_v[buf, e, pl.ds(8, 16)]
                        acc_b += k2 * rows_v[buf, e, pl.ds(40, 16)]
                        acc_b += k3 * rows_v[buf, e, pl.ds(72, 16)]
                        u_v[buf, e, pl.ds(8, 16)] = acc_b
                return carry

            lax.fori_loop(0, CH // 4, edge_body, 0)

        def group_body(g, carry):
            if l1:
                blk = w * nitf + g * GRP
                off = w * ept + g * (GRP * CH)
                pltpu.sync_copy(src_hbm.at[0, pl.ds(blk, GRP), :], isrc_v)
            else:
                blk = s * nitf + g * GRP
                off = s * ept + g * (GRP * CH)
                pltpu.sync_copy(src_hbm.at[c, pl.ds(blk, GRP), :], isrc_v)
            pltpu.sync_copy(dst_hbm.at[pl.ds(blk, GRP), :], idst_v)
            for j in range(3):
                pltpu.sync_copy(kern_hbm.at[j, pl.ds(off, GRP * CH)],
                                kv_v.at[j, pl.ds(0, GRP * CH)])
            g_desc(0).start()
            g_desc(1).start()
            for r in range(GRP):
                g_desc(r).wait()
                if r >= nbuf:
                    s_desc(r - nbuf).wait()
                if r + 2 < GRP:
                    g_desc(r + 2).start()
                combine(r)
                s_desc(r).start(add=True)
            for r in range(GRP - nbuf, GRP):
                s_desc(r).wait()
            return carry

        lax.fori_loop(0, ngrp, group_body, 0)
        plsc.subcore_barrier()
        pltpu.sync_copy(table_sh.at[pl.ds(s * ZR, ZR), :],
                        m_hbm.at[c, pl.ds(s * ZR, ZR), :])

    return fused_k


def _full(shape):
    return pl.BlockSpec(shape, lambda i: (0,) * len(shape))


def _rows(shape):
    return pl.BlockSpec(shape, lambda i: (i,) + (0,) * (len(shape) - 1))


def _rows1(shape):
    return pl.BlockSpec(shape, lambda i: (0, i) + (0,) * (len(shape) - 2))


def _prep_body(x_ref, g9_ref, w_ref, we1h_ref, be1_ref,
               htab_ref, self1_ref, t0_ref):
    xc = [x_ref[:, d:d + 1] for d in range(3)]
    h = [
        g9_ref[:, 3 * i:3 * i + 1] * xc[0]
        + g9_ref[:, 3 * i + 1:3 * i + 2] * xc[1]
        + g9_ref[:, 3 * i + 2:3 * i + 3] * xc[2]
        for i in range(3)
    ]
    nrm = jnp.sqrt(h[0] * h[0] + h[1] * h[1] + h[2] * h[2])
    inv = 1.0 / (nrm + 1e-8)
    hn = [hi * inv for hi in h]
    z5 = jnp.zeros((x_ref.shape[0], 5), F32)
    htab_ref[...] = jnp.concatenate(hn + [z5] + hn + [z5], axis=1)
    self1_ref[...] = (hn[0] * w_ref[0:1, :] + hn[1] * w_ref[1:2, :]
                      + hn[2] * w_ref[2:3, :])
    t0_ref[...] = (be1_ref[...] + hn[0] * we1h_ref[0:1, :]
                   + hn[1] * we1h_ref[1:2, :] + hn[2] * we1h_ref[2:3, :])


def _layer1_body(m0_ref, m1_ref, self1_ref, w1_ref, b1_ref, wz_ref, w2s_ref,
                 h1_ref, ztab_ref, self2_ref):
    pre = self1_ref[...] + b1_ref[...]
    m = m0_ref[...] + m1_ref[...]
    for j in range(3):
        for d in range(3):
            pre += (m[:, 8 * j + d:8 * j + d + 1]
                    * w1_ref[3 + 3 * j + d:4 + 3 * j + d, :])
    h1 = jnp.tanh(pre)
    h1_ref[...] = h1
    z = jnp.dot(h1, wz_ref[...], preferred_element_type=F32)
    ztab_ref[0] = z[:, 0:96]
    ztab_ref[1] = z[:, 96:192]
    self2_ref[...] = jnp.dot(h1, w2s_ref[...], preferred_element_type=F32)


def _final_body(h1_ref, self2_ref, ma_ref, mb_ref, b2_ref,
                we1a_ref, we1b_ref, t0_ref, we2_ref, be2_ref, emb_ref):
    msum = jnp.concatenate([ma_ref[...], mb_ref[...]], axis=1)
    h2 = jnp.tanh(self2_ref[...] + msum + b2_ref[...])
    h1 = h1_ref[...]
    t = t0_ref[...] + jnp.dot(h1, we1a_ref[...], preferred_element_type=F32)
    t += jnp.dot(h2, we1b_ref[...], preferred_element_type=F32)
    a = jnp.maximum(t, 0.0)
    emb_ref[...] = (jnp.dot(a, we2_ref[...], preferred_element_type=F32)
                    + be2_ref[...])


def kernel(x, gauges, edge_index, kernels, W1, b1, W2, b2, We1, be1, We2, be2):
    src = edge_index[0]
    dst = edge_index[1]
    g9 = gauges.reshape(N, 9)
    srcp = jnp.pad(src, (0, E_PAD - E))
    src2 = jnp.stack([srcp, srcp + N]).reshape(2, E_PAD // CH, CH)
    pad_dump = N + (jnp.arange(E_PAD - E, dtype=jnp.int32) % NDUMP)
    dstp = jnp.concatenate([dst, pad_dump]).reshape(E_PAD // CH, CH)
    kpad = jnp.pad(kernels, ((0, 0), (0, E_PAD - E)))
    zeros_tab = jnp.zeros((ZR, 24), F32)
    wz_cols = []
    for h in range(2):
        for j in range(3):
            blk = W2[48 * (j + 1):48 * (j + 2), 24 * h:24 * h + 24]
            wz_cols.append(blk)
            wz_cols.append(jnp.zeros((48, 8), F32))
    Wz = jnp.concatenate(wz_cols, axis=1)
    b1r = b1.reshape(1, 48)
    b2r = b2.reshape(1, 48)
    grid_n = N // NB

    htab, self1, t0 = pl.pallas_call(
        _prep_body,
        grid=(grid_n,),
        in_specs=[_rows((NB, 3)), _rows((NB, 9)), _full((3, 48)),
                  _full((3, 128)), _full((1, 128))],
        out_specs=[_rows((NB, 16)), _rows((NB, 48)), _rows((NB, 128))],
        out_shape=[jax.ShapeDtypeStruct((N, 16), F32),
                   jax.ShapeDtypeStruct((N, 48), F32),
                   jax.ShapeDtypeStruct((N, 128), F32)],
    )(x, g9, W1[0:3], We1[0:3], be1.reshape(1, 128))

    m1 = _make_fused(16, True)(htab, src2, dstp, kpad, zeros_tab)

    h1, ztab, self2 = pl.pallas_call(
        _layer1_body,
        grid=(grid_n,),
        in_specs=[_rows((NB, 24)), _rows((NB, 24)), _rows((NB, 48)),
                  _full((12, 48)), _full((1, 48)), _full((48, 192)),
                  _full((48, 48))],
        out_specs=[_rows((NB, 48)), _rows1((2, NB, 96)), _rows((NB, 48))],
        out_shape=[jax.ShapeDtypeStruct((N, 48), F32),
                   jax.ShapeDtypeStruct((2, N, 96), F32),
                   jax.ShapeDtypeStruct((N, 48), F32)],
    )(m1[0, :N], m1[1, :N], self1, W1, b1r, Wz, W2[0:48])

    m2 = _make_fused(96, False)(ztab.reshape(2 * N, 96), src2, dstp, kpad,
                                zeros_tab)

    emb = pl.pallas_call(
        _final_body,
        grid=(grid_n,),
        in_specs=[_rows((NB, 48)), _rows((NB, 48)), _rows((NB, 24)),
                  _rows((NB, 24)), _full((1, 48)),
                  _full((48, 128)), _full((48, 128)), _rows((NB, 128)),
                  _full((128, 64)), _full((1, 64))],
        out_specs=_rows((NB, 64)),
        out_shape=jax.ShapeDtypeStruct((N, 64), F32),
    )(h1, self2, m2[0, :N], m2[1, :N], b2r,
      We1[3:51], We1[51:99], t0, We2, be2.reshape(1, 64))

    return emb
